# Initial kernel scaffold; baseline (speedup 1.0000x reference)
import jax
import jax.numpy as jnp
from jax.experimental import pallas as pl


def kernel(X, S, mask, chain_M, residue_idx, chain_encoding_all, index, W_feat, b_feat, W_e, b_e, W_s, W_out, b_out, enc_Wl, enc_Wr, enc_We, enc_att, enc_b, dec_Wl, dec_Wr, dec_We, dec_att, dec_b):
    raise NotImplementedError("write your pallas kernel here")


# speedup vs baseline: 21.1772x; 21.1772x over previous
import functools

import jax
import jax.numpy as jnp
from jax import lax
from jax.experimental import pallas as pl
from jax.experimental.pallas import tpu as pltpu
from jax.experimental.pallas import tpu_sc as plsc

L = 1024
K = 32
D = 128
RB = 128
NBLK = L // RB
NW = 16
EPW = (L * K) // NW
CPW = L // NW


def _tc1_body(caT, caC, comboT, comboC, mu_in, w_feat, w_e, dec_we, w_s,
              b_feat, b_e, enc_b, dec_wl, dec_wr, dec_att,
              logit_out, eidx_out, wf3_s, ws2_s, misc_s):
    pid = pl.program_id(0)

    @pl.when(pid == 0)
    def _fold():
        we_top = dec_we[0:128, :]
        we_bot = dec_we[128:256, :]
        m1 = jnp.dot(w_e[...], we_top, preferred_element_type=jnp.float32)
        wf3_s[...] = jnp.dot(w_feat[...], m1, preferred_element_type=jnp.float32)
        ws2_s[...] = jnp.dot(w_s[...], we_bot, preferred_element_type=jnp.float32)
        bconst = (jnp.dot(b_feat[...], m1, preferred_element_type=jnp.float32)
                  + jnp.dot(b_e[...], we_top, preferred_element_type=jnp.float32))
        cvec = (jnp.dot(enc_b[...], dec_wl[...], preferred_element_type=jnp.float32)
                + jnp.dot(enc_b[...], dec_wr[...], preferred_element_type=jnp.float32))
        misc_s[0:1, :] = cvec
        misc_s[1:2, :] = bconst

    xi = caC[:, 0:1]
    yi = caC[:, 1:2]
    zi = caC[:, 2:3]
    mi = caC[:, 3:4]
    xj = caT[0:1, :]
    yj = caT[1:2, :]
    zj = caT[2:3, :]
    mj = caT[3:4, :]
    combo_src = comboC[:, 0:1]
    combo_row = comboT[0:1, :]

    dx = xi - xj
    dy = yi - yj
    dz = zi - zj
    dist = jnp.sqrt(dx * dx + dy * dy + dz * dz + 1e-6)
    cur = dist + (1.0 - mi * mj) * 10000.0

    iota_l = lax.broadcasted_iota(jnp.int32, (RB, L), 1)
    ri_src = combo_src & 2047
    ch_src = (combo_src >> 11) & 3
    pos_src = combo_src >> 18

    cvec = misc_s[0:1, :]
    bconst = misc_s[1:2, :]
    att_v = dec_att[...]
    mu = mu_in[...]
    iota_tok = lax.broadcasted_iota(jnp.int32, (RB, 21), 1)

    lg_cols = []
    idx_cols = []
    for _t in range(K):
        mn = jnp.min(cur, axis=1, keepdims=True)
        is_min = cur == mn
        idxsel = jnp.min(jnp.where(is_min, iota_l, jnp.int32(1 << 30)),
                         axis=1, keepdims=True)
        onehot = iota_l == idxsel
        combo_d = jnp.sum(jnp.where(onehot, combo_row, jnp.int32(0)),
                          axis=1, keepdims=True)
        cur = jnp.where(onehot, jnp.float32(1e30), cur)

        ri_d = combo_d & 2047
        ch_d = (combo_d >> 11) & 3
        tok_d = (combo_d >> 13) & 31
        pos_d = combo_d >> 18

        off = jnp.clip(ri_src - ri_d, -32, 32).astype(jnp.float32) * (1.0 / 32.0)
        same = (ch_src == ch_d).astype(jnp.float32)
        r = (mn - mu) * (1.0 / 1.25)
        rbf = jnp.exp(-(r * r))
        feat = jnp.concatenate([rbf, off, same], axis=1)
        t_edge = jnp.dot(feat, wf3_s[...], preferred_element_type=jnp.float32)
        tok_oh = (iota_tok == tok_d).astype(jnp.float32)
        ws_row = jnp.dot(tok_oh, ws2_s[...], preferred_element_type=jnp.float32)
        att_m = (pos_src > pos_d).astype(jnp.float32) * mi
        u = cvec + mi * (t_edge + bconst) + att_m * ws_row
        lr = jnp.where(u >= 0, u, 0.2 * u)
        lg = jnp.sum(lr * att_v, axis=1, keepdims=True)
        lg_cols.append(lg)
        idx_cols.append(idxsel)

    logit_out[...] = jnp.concatenate(lg_cols, axis=1)
    eidx_out[...] = jnp.concatenate(idx_cols, axis=1)


def _tc1_call(caT, caC, comboT, comboC, mu_in, w_feat, w_e, dec_we, w_s,
              b_feat, b_e, enc_b, dec_wl, dec_wr, dec_att):
    full = lambda s: pl.BlockSpec(s, lambda i: (0, 0))
    rows = lambda s: pl.BlockSpec(s, lambda i: (i, 0))
    return pl.pallas_call(
        _tc1_body,
        grid=(NBLK,),
        in_specs=[
            full((8, L)), rows((RB, 8)), full((8, L)), rows((RB, 8)),
            full((1, 16)), full((18, D)), full((D, D)), full((2 * D, D)),
            full((21, D)), full((1, D)), full((1, D)), full((1, D)),
            full((D, D)), full((D, D)), full((1, D)),
        ],
        out_specs=[rows((RB, K)), rows((RB, K))],
        out_shape=[
            jax.ShapeDtypeStruct((L, K), jnp.float32),
            jax.ShapeDtypeStruct((L, K), jnp.int32),
        ],
        scratch_shapes=[
            pltpu.VMEM((18, D), jnp.float32),
            pltpu.VMEM((21, D), jnp.float32),
            pltpu.VMEM((8, D), jnp.float32),
        ],
    )(caT, caC, comboT, comboC, mu_in, w_feat, w_e, dec_we, w_s,
      b_feat, b_e, enc_b, dec_wl, dec_wr, dec_att)


def _sc_body(dst_hbm, lg_hbm, out_hbm, idx_v, lg_v, acc, red, slab,
             gm_loc, m_all, sh_part, sh_comb, sem):
    sid = lax.axis_index("s")
    base_e = sid * EPW
    pltpu.sync_copy(dst_hbm.at[pl.ds(base_e, EPW)], idx_v)
    pltpu.sync_copy(lg_hbm.at[pl.ds(base_e, EPW)], lg_v)
    lane_base = lax.iota(jnp.int32, 16) * L

    def _init(val):
        def body(i, _):
            acc[pl.ds(i * 16, 16)] = jnp.full((16,), val, jnp.float32)
            return 0
        lax.fori_loop(0, (16 * L) // 16, body, 0)

    def _lane_reduce(op):
        def body(c, _):
            m = acc[pl.ds(c * 16, 16)]
            for r in range(1, 16):
                m = op(m, acc[pl.ds(r * L + c * 16, 16)])
            red[pl.ds(c * 16, 16)] = m
            return 0
        lax.fori_loop(0, L // 16, body, 0)

    def _slab_fetch():
        for r in range(16):
            pltpu.sync_copy(sh_part.at[r, pl.ds(sid * CPW, CPW)], slab.at[r])

    _init(-1e30)

    def max_body(j, _):
        idx16 = idx_v[pl.ds(j * 16, 16)]
        lg16 = lg_v[pl.ds(j * 16, 16)]
        fidx = lane_base + idx16
        curv = plsc.load_gather(acc, [fidx])
        plsc.store_scatter(acc, [fidx], jnp.maximum(curv, lg16))
        return 0
    lax.fori_loop(0, EPW // 16, max_body, 0)
    _lane_reduce(jnp.maximum)
    pltpu.sync_copy(red, sh_part.at[sid])
    plsc.subcore_barrier()
    _slab_fetch()
    for cc in range(CPW // 16):
        m = slab[0, pl.ds(cc * 16, 16)]
        for r in range(1, 16):
            m = jnp.maximum(m, slab[r, pl.ds(cc * 16, 16)])
        gm_loc[pl.ds(cc * 16, 16)] = m
    pltpu.sync_copy(gm_loc, sh_comb.at[pl.ds(sid * CPW, CPW)])
    plsc.subcore_barrier()
    pltpu.sync_copy(sh_comb, m_all)

    _init(0.0)

    def sum_body(j, _):
        idx16 = idx_v[pl.ds(j * 16, 16)]
        lg16 = lg_v[pl.ds(j * 16, 16)]
        mg = plsc.load_gather(m_all, [idx16])
        ex = jnp.exp(lg16 - mg)
        fidx = lane_base + idx16
        curv = plsc.load_gather(acc, [fidx])
        plsc.store_scatter(acc, [fidx], curv + ex)
        return 0
    lax.fori_loop(0, EPW // 16, sum_body, 0)
    _lane_reduce(jnp.add)
    plsc.subcore_barrier()
    pltpu.sync_copy(red, sh_part.at[sid])
    plsc.subcore_barrier()
    _slab_fetch()
    for cc in range(CPW // 16):
        den = slab[0, pl.ds(cc * 16, 16)]
        for r in range(1, 16):
            den = den + slab[r, pl.ds(cc * 16, 16)]
        gm_loc[pl.ds(cc * 16, 16)] = den / (den + 1e-16)
    pltpu.sync_copy(gm_loc, out_hbm.at[pl.ds(sid * CPW, CPW)])


def _sc_call(dst_flat, lg_flat):
    mesh = plsc.VectorSubcoreMesh(core_axis_name="c", subcore_axis_name="s",
                                  num_cores=1)
    fn = functools.partial(
        pl.kernel,
        mesh=mesh,
        compiler_params=pltpu.CompilerParams(needs_layout_passes=False),
        out_type=jax.ShapeDtypeStruct((L,), jnp.float32),
        scratch_types=[
            pltpu.VMEM((EPW,), jnp.int32),
            pltpu.VMEM((EPW,), jnp.float32),
            pltpu.VMEM((16 * L,), jnp.float32),
            pltpu.VMEM((L,), jnp.float32),
            pltpu.VMEM((16, CPW), jnp.float32),
            pltpu.VMEM((CPW,), jnp.float32),
            pltpu.VMEM((L,), jnp.float32),
            pltpu.VMEM_SHARED((16, L), jnp.float32),
            pltpu.VMEM_SHARED((L,), jnp.float32),
            pltpu.SemaphoreType.DMA,
        ],
    )(_sc_body)
    return fn(dst_flat, lg_flat)


def _tc2_body(s_ref, enc_b, dec_wl, dec_b, w_out, b_out, out_ref):
    v = jnp.dot(enc_b[...], dec_wl[...], preferred_element_type=jnp.float32)
    on = s_ref[...] * v + dec_b[...]
    lg = jnp.dot(on, w_out[...], preferred_element_type=jnp.float32) + b_out[...]
    mx = jnp.max(lg, axis=1, keepdims=True)
    ex = jnp.exp(lg - mx)
    out_ref[...] = lg - mx - jnp.log(jnp.sum(ex, axis=1, keepdims=True))


def _tc2_call(s_col, enc_b, dec_wl, dec_b, w_out, b_out):
    return pl.pallas_call(
        _tc2_body,
        out_shape=jax.ShapeDtypeStruct((L, 21), jnp.float32),
    )(s_col, enc_b, dec_wl, dec_b, w_out, b_out)


def kernel(X, S, mask, chain_M, residue_idx, chain_encoding_all, index,
           W_feat, b_feat, W_e, b_e, W_s, W_out, b_out,
           enc_Wl, enc_Wr, enc_We, enc_att, enc_b,
           dec_Wl, dec_Wr, dec_We, dec_att, dec_b):
    f32 = jnp.float32
    noise = 0.1 * jax.random.normal(jax.random.key(42), X.shape, dtype=X.dtype)
    Ca = (X + noise)[index][:, 1, :]
    mask1 = mask[index].astype(f32)
    cm = chain_M[index].astype(f32) * mask1
    dnoise = jax.random.normal(jax.random.key(7), cm.shape)
    decoding_order = jnp.argsort((cm + 0.0001) * jnp.abs(dnoise))
    pos = jnp.argsort(decoding_order).astype(jnp.int32)
    ri = residue_idx[index].astype(jnp.int32)
    ch = chain_encoding_all[index].astype(jnp.int32)
    S1 = S[index].astype(jnp.int32)
    combo = ri | (ch << 11) | (S1 << 13) | (pos << 18)

    caT = jnp.concatenate(
        [Ca.T.astype(f32), mask1[None, :], jnp.zeros((4, L), f32)], axis=0)
    caC = caT.T
    comboT = jnp.concatenate([combo[None, :], jnp.zeros((7, L), jnp.int32)], 0)
    comboC = comboT.T
    mu_in = jnp.linspace(2.0, 22.0, 16).astype(f32).reshape(1, 16)

    r2 = lambda a: a.reshape(1, D).astype(f32)
    logit, eidx = _tc1_call(
        caT, caC, comboT, comboC, mu_in,
        W_feat.astype(f32), W_e.astype(f32), dec_We.astype(f32),
        W_s.astype(f32), r2(b_feat), r2(b_e), r2(enc_b),
        dec_Wl.astype(f32), dec_Wr.astype(f32), r2(dec_att))

    s = _sc_call(eidx.reshape(L * K), logit.reshape(L * K))

    out = _tc2_call(s.reshape(L, 1), r2(enc_b), dec_Wl.astype(f32),
                    r2(dec_b), W_out.astype(f32),
                    b_out.reshape(1, 21).astype(f32))
    return out
